# baseline (speedup 1.0000x reference)
import functools

import jax
import jax.numpy as jnp
from jax import lax
from jax.experimental import pallas as pl
from jax.experimental.pallas import tpu as pltpu
from jax.experimental.pallas import tpu_sc as plsc

_B = 16384
_EMB = 32
_NW = 32
_BPW = _B // _NW
_CHUNK = 128
_NCH = _BPW // _CHUNK


def _sc_body(users_hbm, items_hbm, utab_hbm, itab_hbm, bias_hbm, out_hbm,
             uidx, iidx_raw, iidx_clean, urows, irows, bias_v, out_v, sem):
    wid = lax.axis_index("s") * 2 + lax.axis_index("c")
    base = wid * _BPW

    for j in range(_NCH):
        pltpu.sync_copy(users_hbm.at[pl.ds(base + j * _CHUNK, _CHUNK)],
                        uidx.at[j])
    pltpu.sync_copy(items_hbm.at[pl.ds(base, _BPW)], iidx_raw)

    for c in range(_BPW // 16):
        v = iidx_raw[pl.ds(c * 16, 16)]
        cv = jnp.where(v == jnp.int32(-1), jnp.int32(0), v)
        iidx_clean[c // (_CHUNK // 16), pl.ds((c % (_CHUNK // 16)) * 16, 16)] = cv

    copies = []
    for j in range(_NCH):
        sl = pl.ds(j * _CHUNK, _CHUNK)
        copies.append(pltpu.make_async_copy(
            utab_hbm.at[uidx.at[j]], urows.at[sl], sem))
        copies.append(pltpu.make_async_copy(
            itab_hbm.at[iidx_clean.at[j]], irows.at[sl], sem))
        copies.append(pltpu.make_async_copy(
            bias_hbm.at[iidx_clean.at[j]], bias_v.at[sl], sem))
    for cp in copies:
        cp.start()
    for cp in copies:
        cp.wait()

    lane = lax.iota(jnp.int32, 16)
    cols = [(lane + t) % _EMB for t in range(_EMB)]

    def chunk(c, carry):
        rows16 = c * 16 + lane
        acc = bias_v[pl.ds(c * 16, 16)]
        for t in range(_EMB):
            u = plsc.load_gather(urows, [rows16, cols[t]])
            v = plsc.load_gather(irows, [rows16, cols[t]])
            acc = acc + u * v
        it = iidx_raw[pl.ds(c * 16, 16)]
        res = jnp.where(it == jnp.int32(-1), jnp.float32(-100.0), acc)
        out_v[pl.ds(c * 16, 16)] = res
        return carry

    lax.fori_loop(0, _BPW // 16, chunk, 0)

    pltpu.sync_copy(out_v, out_hbm.at[pl.ds(base, _BPW)])


@jax.jit
def _sc_call(users, items, user_table, item_table, bias_flat):
    mesh = plsc.VectorSubcoreMesh(core_axis_name="c", subcore_axis_name="s")
    f = pl.kernel(
        _sc_body,
        mesh=mesh,
        compiler_params=pltpu.CompilerParams(
            needs_layout_passes=False, use_tc_tiling_on_sc=False),
        out_type=jax.ShapeDtypeStruct((_B,), jnp.float32),
        scratch_types=[
            pltpu.VMEM((_NCH, _CHUNK), jnp.int32),
            pltpu.VMEM((_BPW,), jnp.int32),
            pltpu.VMEM((_NCH, _CHUNK), jnp.int32),
            pltpu.VMEM((_BPW, _EMB), jnp.float32),
            pltpu.VMEM((_BPW, _EMB), jnp.float32),
            pltpu.VMEM((_BPW,), jnp.float32),
            pltpu.VMEM((_BPW,), jnp.float32),
            pltpu.SemaphoreType.DMA,
        ],
    )
    return f(users, items, user_table, item_table, bias_flat)


def kernel(users, items, user_table, item_table, bias_table):
    users32 = users.astype(jnp.int32)
    items32 = items.astype(jnp.int32)
    bias_flat = jnp.reshape(bias_table, (-1,))
    return _sc_call(users32, items32, user_table, item_table, bias_flat)
